# baseline (speedup 1.0000x reference)
import functools
import math

import jax
import jax.numpy as jnp
from jax import lax
from jax.experimental import pallas as pl
from jax.experimental.pallas import tpu as pltpu
from jax.experimental.pallas import tpu_sc as plsc

BATCH = 1024
MAX_LEN = 200
D_MODEL = 128
HALF = MAX_LEN // 2
NUM_WORKERS = 32
SEQ_PER_WORKER = BATCH // NUM_WORKERS
SCALE = math.sqrt(float(D_MODEL))
LANES = 16
GROUPS = D_MODEL // LANES


def _fire_gathers(table_hbm, idx_v, rows_v, sem):
  pltpu.async_copy(table_hbm.at[idx_v.at[0]], rows_v.at[pl.ds(0, HALF)], sem)
  pltpu.async_copy(table_hbm.at[idx_v.at[1]], rows_v.at[pl.ds(HALF, HALF)],
                   sem)


def _wait_gathers(table_hbm, idx_v, rows_v, sem):
  pltpu.make_async_copy(table_hbm.at[idx_v.at[0]],
                        rows_v.at[pl.ds(0, HALF)], sem).wait()
  pltpu.make_async_copy(table_hbm.at[idx_v.at[1]],
                        rows_v.at[pl.ds(HALF, HALF)], sem).wait()


def _sc_kernel(table_hbm, idx_hbm, pos_hbm, out_hbm, idx_v0, idx_v1,
               rows_v0, rows_v1, pos_v, sem0, sem1):
  wid = lax.axis_index("c") * 16 + lax.axis_index("s")
  base = wid * SEQ_PER_WORKER

  pltpu.sync_copy(pos_hbm, pos_v)

  idx_bufs = (idx_v0, idx_v1)
  rows_bufs = (rows_v0, rows_v1)
  sems = (sem0, sem1)

  pltpu.sync_copy(idx_hbm.at[base], idx_v0)
  _fire_gathers(table_hbm, idx_v0, rows_v0, sem0)

  scale = jnp.float32(SCALE)

  def chunk_pair(c2, _):
    for b in range(2):
      c = c2 * 2 + b
      cur_idx, cur_rows, cur_sem = idx_bufs[b], rows_bufs[b], sems[b]
      nxt_idx, nxt_rows, nxt_sem = (idx_bufs[1 - b], rows_bufs[1 - b],
                                    sems[1 - b])

      @pl.when(c < SEQ_PER_WORKER - 1)
      def _():
        pltpu.sync_copy(idx_hbm.at[base + c + 1], nxt_idx)
        _fire_gathers(table_hbm, nxt_idx, nxt_rows, nxt_sem)

      _wait_gathers(table_hbm, cur_idx, cur_rows, cur_sem)

      def fma_row(i, _):
        for k in range(GROUPS):
          sl = pl.ds(k * LANES, LANES)
          cur_rows[i, sl] = cur_rows[i, sl] * scale + pos_v[i, sl]
        return 0

      lax.fori_loop(0, MAX_LEN, fma_row, 0, unroll=2)

      pltpu.sync_copy(cur_rows, out_hbm.at[base + c])
    return 0

  lax.fori_loop(0, SEQ_PER_WORKER // 2, chunk_pair, 0)


@jax.jit
def _run(table, idx3, pos2d):
  mesh = plsc.VectorSubcoreMesh(core_axis_name="c", subcore_axis_name="s")
  f = pl.kernel(
      _sc_kernel,
      out_type=jax.ShapeDtypeStruct((BATCH, MAX_LEN, D_MODEL), jnp.float32),
      mesh=mesh,
      scratch_types=[
          pltpu.VMEM((2, HALF), jnp.int32),
          pltpu.VMEM((2, HALF), jnp.int32),
          pltpu.VMEM((MAX_LEN, D_MODEL), jnp.float32),
          pltpu.VMEM((MAX_LEN, D_MODEL), jnp.float32),
          pltpu.VMEM((MAX_LEN, D_MODEL), jnp.float32),
          pltpu.SemaphoreType.DMA,
          pltpu.SemaphoreType.DMA,
      ],
  )
  return f(table, idx3, pos2d)


def kernel(x, table, pos_enc):
  idx3 = x.reshape(BATCH, 2, HALF)
  pos2d = pos_enc.reshape(MAX_LEN, D_MODEL)
  return _run(table, idx3, pos2d)


# speedup vs baseline: 1.5703x; 1.5703x over previous
import math

import jax
import jax.numpy as jnp
from jax import lax
from jax.experimental import pallas as pl
from jax.experimental.pallas import tpu as pltpu
from jax.experimental.pallas import tpu_sc as plsc

BATCH = 1024
MAX_LEN = 200
D_MODEL = 128
CHUNK = 100
NUM_WORKERS = 32
CHUNKS_PER_WORKER = BATCH * MAX_LEN // (NUM_WORKERS * CHUNK)
NBUF = 4
SCALE = math.sqrt(float(D_MODEL))
LANES = 16
GROUPS = D_MODEL // LANES


def _out_slice(out_hbm, cbase, c):
  return out_hbm.at[cbase + c]


def _sc_kernel(table_hbm, idx_hbm, pos_hbm, out_hbm, idx_v, rows_v0, rows_v1,
               rows_v2, rows_v3, pos_v, gsem0, gsem1, gsem2, gsem3, osem0,
               osem1, osem2, osem3):
  wid = lax.axis_index("c") * 16 + lax.axis_index("s")
  cbase = wid * CHUNKS_PER_WORKER

  rows = (rows_v0, rows_v1, rows_v2, rows_v3)
  gsems = (gsem0, gsem1, gsem2, gsem3)
  osems = (osem0, osem1, osem2, osem3)

  pltpu.sync_copy(idx_hbm.at[pl.ds(cbase, CHUNKS_PER_WORKER)], idx_v)
  pltpu.sync_copy(pos_hbm, pos_v)

  def fire(c, buf):
    pltpu.async_copy(table_hbm.at[idx_v.at[c]], rows[buf], gsems[buf])

  def wait_gather(c, buf):
    pltpu.make_async_copy(table_hbm.at[idx_v.at[c]], rows[buf],
                          gsems[buf]).wait()

  fire(0, 0)
  fire(1, 1)

  scale = jnp.float32(SCALE)

  def quad(c4, _):
    for b in range(NBUF):
      c = c4 * NBUF + b
      half = b % 2
      cur = rows[b]

      nb = (b + 2) % NBUF

      @pl.when(c + 2 < CHUNKS_PER_WORKER)
      def _():
        @pl.when(c >= 2)
        def _():
          pltpu.make_async_copy(rows[nb], _out_slice(out_hbm, cbase, c - 2),
                                osems[nb]).wait()

        fire(c + 2, nb)

      wait_gather(c, b)

      pos_off = half * CHUNK

      @plsc.parallel_loop(0, CHUNK, unroll=2)
      def _(i):
        for k in range(GROUPS):
          sl = pl.ds(k * LANES, LANES)
          cur[i, sl] = cur[i, sl] * scale + pos_v[pos_off + i, sl]

      pltpu.async_copy(cur, _out_slice(out_hbm, cbase, c), osems[b])
    return 0

  lax.fori_loop(0, CHUNKS_PER_WORKER // NBUF, quad, 0)

  for j in range(CHUNKS_PER_WORKER - NBUF, CHUNKS_PER_WORKER):
    b = j % NBUF
    pltpu.make_async_copy(rows[b], _out_slice(out_hbm, cbase, j),
                          osems[b]).wait()


@jax.jit
def _run(table, idx2, pos2d):
  mesh = plsc.VectorSubcoreMesh(core_axis_name="c", subcore_axis_name="s")
  f = pl.kernel(
      _sc_kernel,
      out_type=jax.ShapeDtypeStruct(
          (BATCH * MAX_LEN // CHUNK, CHUNK, D_MODEL), jnp.float32),
      mesh=mesh,
      scratch_types=[
          pltpu.VMEM((CHUNKS_PER_WORKER, CHUNK), jnp.int32),
          pltpu.VMEM((CHUNK, D_MODEL), jnp.float32),
          pltpu.VMEM((CHUNK, D_MODEL), jnp.float32),
          pltpu.VMEM((CHUNK, D_MODEL), jnp.float32),
          pltpu.VMEM((CHUNK, D_MODEL), jnp.float32),
          pltpu.VMEM((MAX_LEN, D_MODEL), jnp.float32),
          pltpu.SemaphoreType.DMA,
          pltpu.SemaphoreType.DMA,
          pltpu.SemaphoreType.DMA,
          pltpu.SemaphoreType.DMA,
          pltpu.SemaphoreType.DMA,
          pltpu.SemaphoreType.DMA,
          pltpu.SemaphoreType.DMA,
          pltpu.SemaphoreType.DMA,
      ],
  )
  return f(table, idx2, pos2d)


def kernel(x, table, pos_enc):
  idx2 = x.reshape(BATCH * MAX_LEN // CHUNK, CHUNK)
  pos2d = pos_enc.reshape(MAX_LEN, D_MODEL)
  out = _run(table, idx2, pos2d)
  return out.reshape(BATCH, MAX_LEN, D_MODEL)


# speedup vs baseline: 2.9351x; 1.8691x over previous
import math

import jax
import jax.numpy as jnp
from jax import lax
from jax.experimental import pallas as pl
from jax.experimental.pallas import tpu as pltpu
from jax.experimental.pallas import tpu_sc as plsc

BATCH = 1024
MAX_LEN = 200
D_MODEL = 128
CHUNK = 128
NUM_WORKERS = 32
NUM_CHUNKS = BATCH * MAX_LEN // CHUNK
CHUNKS_PER_WORKER = NUM_CHUNKS // NUM_WORKERS
NBUF = 4
MAIN_CHUNKS = CHUNKS_PER_WORKER - (CHUNKS_PER_WORKER % NBUF)
SCALE = math.sqrt(float(D_MODEL))
LANES = 16
GROUPS = D_MODEL // LANES


def _sc_kernel(table_hbm, idx_hbm, pos_hbm, out_hbm, idx_v, rows_v0, rows_v1,
               rows_v2, rows_v3, pos_v, gsem0, gsem1, gsem2, gsem3, osem0,
               osem1, osem2, osem3):
  wid = lax.axis_index("c") * 16 + lax.axis_index("s")
  cbase = wid * CHUNKS_PER_WORKER

  rows = (rows_v0, rows_v1, rows_v2, rows_v3)
  gsems = (gsem0, gsem1, gsem2, gsem3)
  osems = (osem0, osem1, osem2, osem3)

  pltpu.sync_copy(idx_hbm.at[wid], idx_v)
  pltpu.sync_copy(pos_hbm, pos_v)

  def fire(c, buf):
    pltpu.async_copy(table_hbm.at[idx_v.at[c]], rows[buf], gsems[buf])

  def wait_gather(c, buf):
    pltpu.make_async_copy(table_hbm.at[idx_v.at[c]], rows[buf],
                          gsems[buf]).wait()

  def wait_out(c, buf):
    pltpu.make_async_copy(rows[buf], out_hbm.at[cbase + c],
                          osems[buf]).wait()

  fire(0, 0)
  fire(1, 1)

  scale = jnp.float32(SCALE)

  def chunk_body(c, b):
    cur = rows[b]
    nb = (b + 2) % NBUF

    @pl.when(c + 2 < CHUNKS_PER_WORKER)
    def _():
      @pl.when(c >= 2)
      def _():
        wait_out(c - 2, nb)

      fire(c + 2, nb)

    wait_gather(c, b)

    pos_off = lax.rem(c * CHUNK, MAX_LEN)

    @plsc.parallel_loop(0, CHUNK, unroll=2)
    def _(i):
      for k in range(GROUPS):
        sl = pl.ds(k * LANES, LANES)
        cur[i, sl] = cur[i, sl] * scale + pos_v[pos_off + i, sl]

    pltpu.async_copy(cur, out_hbm.at[cbase + c], osems[b])

  def quad(c4, _):
    for b in range(NBUF):
      chunk_body(c4 * NBUF + b, b)
    return 0

  lax.fori_loop(0, MAIN_CHUNKS // NBUF, quad, 0)

  for c in range(MAIN_CHUNKS, CHUNKS_PER_WORKER):
    chunk_body(jnp.int32(c), c % NBUF)

  for j in range(CHUNKS_PER_WORKER - NBUF, CHUNKS_PER_WORKER):
    wait_out(j, j % NBUF)


@jax.jit
def _run(table, idx2, pos2x):
  mesh = plsc.VectorSubcoreMesh(core_axis_name="c", subcore_axis_name="s")
  f = pl.kernel(
      _sc_kernel,
      out_type=jax.ShapeDtypeStruct((NUM_CHUNKS, CHUNK, D_MODEL),
                                    jnp.float32),
      mesh=mesh,
      scratch_types=[
          pltpu.VMEM((CHUNKS_PER_WORKER, CHUNK), jnp.int32),
          pltpu.VMEM((CHUNK, D_MODEL), jnp.float32),
          pltpu.VMEM((CHUNK, D_MODEL), jnp.float32),
          pltpu.VMEM((CHUNK, D_MODEL), jnp.float32),
          pltpu.VMEM((CHUNK, D_MODEL), jnp.float32),
          pltpu.VMEM((2 * MAX_LEN, D_MODEL), jnp.float32),
          pltpu.SemaphoreType.DMA,
          pltpu.SemaphoreType.DMA,
          pltpu.SemaphoreType.DMA,
          pltpu.SemaphoreType.DMA,
          pltpu.SemaphoreType.DMA,
          pltpu.SemaphoreType.DMA,
          pltpu.SemaphoreType.DMA,
          pltpu.SemaphoreType.DMA,
      ],
  )
  return f(table, idx2, pos2x)


def kernel(x, table, pos_enc):
  idx2 = x.reshape(NUM_WORKERS, CHUNKS_PER_WORKER, CHUNK)
  pos2d = pos_enc.reshape(MAX_LEN, D_MODEL)
  pos2x = jnp.concatenate([pos2d, pos2d], axis=0)
  out = _run(table, idx2, pos2x)
  return out.reshape(BATCH, MAX_LEN, D_MODEL)


# speedup vs baseline: 2.9856x; 1.0172x over previous
import math

import jax
import jax.numpy as jnp
from jax import lax
from jax.experimental import pallas as pl
from jax.experimental.pallas import tpu as pltpu
from jax.experimental.pallas import tpu_sc as plsc

BATCH = 1024
MAX_LEN = 200
D_MODEL = 128
CHUNK = 128
NUM_WORKERS = 32
NUM_CHUNKS = BATCH * MAX_LEN // CHUNK
CHUNKS_PER_WORKER = NUM_CHUNKS // NUM_WORKERS
NBUF = 4
MAIN_CHUNKS = CHUNKS_PER_WORKER - (CHUNKS_PER_WORKER % NBUF)
SCALE = math.sqrt(float(D_MODEL))
LANES = 16
GROUPS = D_MODEL // LANES


def _sc_kernel(table_hbm, idx_hbm, pos_hbm, out_hbm, idx_v, rows_v0, rows_v1,
               rows_v2, rows_v3, pos_v, gsem0, gsem1, gsem2, gsem3, osem0,
               osem1, osem2, osem3):
  wid = lax.axis_index("c") * 16 + lax.axis_index("s")
  cbase = wid * CHUNKS_PER_WORKER

  rows = (rows_v0, rows_v1, rows_v2, rows_v3)
  gsems = (gsem0, gsem1, gsem2, gsem3)
  osems = (osem0, osem1, osem2, osem3)

  pltpu.sync_copy(idx_hbm.at[wid], idx_v)
  pltpu.sync_copy(pos_hbm, pos_v.at[pl.ds(0, MAX_LEN)])
  pltpu.sync_copy(pos_hbm, pos_v.at[pl.ds(MAX_LEN, MAX_LEN)])

  def fire(c, buf):
    pltpu.async_copy(table_hbm.at[idx_v.at[c]], rows[buf], gsems[buf])

  def wait_gather(c, buf):
    pltpu.make_async_copy(table_hbm.at[idx_v.at[c]], rows[buf],
                          gsems[buf]).wait()

  def wait_out(c, buf):
    pltpu.make_async_copy(rows[buf], out_hbm.at[cbase + c],
                          osems[buf]).wait()

  fire(0, 0)
  fire(1, 1)

  scale = jnp.float32(SCALE)

  def chunk_body(c, b):
    cur = rows[b]
    nb = (b + 2) % NBUF

    @pl.when(c + 2 < CHUNKS_PER_WORKER)
    def _():
      @pl.when(c >= 2)
      def _():
        wait_out(c - 2, nb)

      fire(c + 2, nb)

    wait_gather(c, b)

    pos_off = lax.rem(c * CHUNK, MAX_LEN)

    @plsc.parallel_loop(0, CHUNK, unroll=2)
    def _(i):
      for k in range(GROUPS):
        sl = pl.ds(k * LANES, LANES)
        cur[i, sl] = cur[i, sl] * scale + pos_v[pos_off + i, sl]

    pltpu.async_copy(cur, out_hbm.at[cbase + c], osems[b])

  def quad(c4, _):
    for b in range(NBUF):
      chunk_body(c4 * NBUF + b, b)
    return 0

  lax.fori_loop(0, MAIN_CHUNKS // NBUF, quad, 0)

  for c in range(MAIN_CHUNKS, CHUNKS_PER_WORKER):
    chunk_body(jnp.int32(c), c % NBUF)

  for j in range(CHUNKS_PER_WORKER - NBUF, CHUNKS_PER_WORKER):
    wait_out(j, j % NBUF)


@jax.jit
def _run(table, idx2, pos2d):
  mesh = plsc.VectorSubcoreMesh(core_axis_name="c", subcore_axis_name="s")
  f = pl.kernel(
      _sc_kernel,
      out_type=jax.ShapeDtypeStruct((NUM_CHUNKS, CHUNK, D_MODEL),
                                    jnp.float32),
      mesh=mesh,
      scratch_types=[
          pltpu.VMEM((CHUNKS_PER_WORKER, CHUNK), jnp.int32),
          pltpu.VMEM((CHUNK, D_MODEL), jnp.float32),
          pltpu.VMEM((CHUNK, D_MODEL), jnp.float32),
          pltpu.VMEM((CHUNK, D_MODEL), jnp.float32),
          pltpu.VMEM((CHUNK, D_MODEL), jnp.float32),
          pltpu.VMEM((2 * MAX_LEN, D_MODEL), jnp.float32),
          pltpu.SemaphoreType.DMA,
          pltpu.SemaphoreType.DMA,
          pltpu.SemaphoreType.DMA,
          pltpu.SemaphoreType.DMA,
          pltpu.SemaphoreType.DMA,
          pltpu.SemaphoreType.DMA,
          pltpu.SemaphoreType.DMA,
          pltpu.SemaphoreType.DMA,
      ],
  )
  return f(table, idx2, pos2d)


def kernel(x, table, pos_enc):
  idx2 = x.reshape(NUM_WORKERS, CHUNKS_PER_WORKER, CHUNK)
  pos2d = pos_enc.reshape(MAX_LEN, D_MODEL)
  out = _run(table, idx2, pos2d)
  return out.reshape(BATCH, MAX_LEN, D_MODEL)


# speedup vs baseline: 3.0203x; 1.0116x over previous
import math

import jax
import jax.numpy as jnp
from jax import lax
from jax.experimental import pallas as pl
from jax.experimental.pallas import tpu as pltpu
from jax.experimental.pallas import tpu_sc as plsc

BATCH = 1024
MAX_LEN = 200
D_MODEL = 128
CHUNK = 128
NUM_WORKERS = 32
NUM_CHUNKS = BATCH * MAX_LEN // CHUNK
CHUNKS_PER_WORKER = NUM_CHUNKS // NUM_WORKERS
NBUF = 5
FIRE_AHEAD = 3
POS_ROWS = 320
SCALE = math.sqrt(float(D_MODEL))
LANES = 16
GROUPS = D_MODEL // LANES


def _sc_kernel(table_hbm, idx_hbm, pos_hbm, out_hbm, idx_v, rows_v0, rows_v1,
               rows_v2, rows_v3, rows_v4, pos_v, gsem0, gsem1, gsem2, gsem3,
               gsem4, osem0, osem1, osem2, osem3, osem4, psem):
  wid = lax.axis_index("c") * 16 + lax.axis_index("s")
  cbase = wid * CHUNKS_PER_WORKER

  rows = (rows_v0, rows_v1, rows_v2, rows_v3, rows_v4)
  gsems = (gsem0, gsem1, gsem2, gsem3, gsem4)
  osems = (osem0, osem1, osem2, osem3, osem4)

  pltpu.sync_copy(idx_hbm.at[wid], idx_v)

  def fire(c, buf):
    pltpu.async_copy(table_hbm.at[idx_v.at[c]], rows[buf], gsems[buf])

  def wait_gather(c, buf):
    pltpu.make_async_copy(table_hbm.at[idx_v.at[c]], rows[buf],
                          gsems[buf]).wait()

  def wait_out(c, buf):
    pltpu.make_async_copy(rows[buf], out_hbm.at[cbase + c],
                          osems[buf]).wait()

  for c0 in range(FIRE_AHEAD):
    fire(c0, c0)

  pos_a = pltpu.async_copy(pos_hbm, pos_v.at[pl.ds(0, MAX_LEN)], psem)
  pos_b = pltpu.async_copy(pos_hbm.at[pl.ds(0, POS_ROWS - MAX_LEN)],
                           pos_v.at[pl.ds(MAX_LEN, POS_ROWS - MAX_LEN)],
                           psem)
  pos_a.wait()
  pos_b.wait()

  scale = jnp.float32(SCALE)

  def chunk_body(c, b):
    cur = rows[b]
    nb = (b + FIRE_AHEAD) % NBUF

    @pl.when(c + FIRE_AHEAD < CHUNKS_PER_WORKER)
    def _():
      @pl.when(c >= NBUF - FIRE_AHEAD)
      def _():
        wait_out(c - (NBUF - FIRE_AHEAD), nb)

      fire(c + FIRE_AHEAD, nb)

    wait_gather(c, b)

    pos_off = lax.rem(c * CHUNK, MAX_LEN)

    @plsc.parallel_loop(0, CHUNK, unroll=2)
    def _(i):
      for k in range(GROUPS):
        sl = pl.ds(k * LANES, LANES)
        cur[i, sl] = cur[i, sl] * scale + pos_v[pos_off + i, sl]

    pltpu.async_copy(cur, out_hbm.at[cbase + c], osems[b])

  def quint(cq, _):
    for b in range(NBUF):
      chunk_body(cq * NBUF + b, b)
    return 0

  lax.fori_loop(0, CHUNKS_PER_WORKER // NBUF, quint, 0)

  for j in range(CHUNKS_PER_WORKER - NBUF, CHUNKS_PER_WORKER):
    wait_out(j, j % NBUF)


@jax.jit
def _run(table, idx2, pos2d):
  mesh = plsc.VectorSubcoreMesh(core_axis_name="c", subcore_axis_name="s")
  f = pl.kernel(
      _sc_kernel,
      out_type=jax.ShapeDtypeStruct((NUM_CHUNKS, CHUNK, D_MODEL),
                                    jnp.float32),
      mesh=mesh,
      scratch_types=[
          pltpu.VMEM((CHUNKS_PER_WORKER, CHUNK), jnp.int32),
          pltpu.VMEM((CHUNK, D_MODEL), jnp.float32),
          pltpu.VMEM((CHUNK, D_MODEL), jnp.float32),
          pltpu.VMEM((CHUNK, D_MODEL), jnp.float32),
          pltpu.VMEM((CHUNK, D_MODEL), jnp.float32),
          pltpu.VMEM((CHUNK, D_MODEL), jnp.float32),
          pltpu.VMEM((POS_ROWS, D_MODEL), jnp.float32),
          pltpu.SemaphoreType.DMA,
          pltpu.SemaphoreType.DMA,
          pltpu.SemaphoreType.DMA,
          pltpu.SemaphoreType.DMA,
          pltpu.SemaphoreType.DMA,
          pltpu.SemaphoreType.DMA,
          pltpu.SemaphoreType.DMA,
          pltpu.SemaphoreType.DMA,
          pltpu.SemaphoreType.DMA,
          pltpu.SemaphoreType.DMA,
          pltpu.SemaphoreType.DMA,
      ],
  )
  return f(table, idx2, pos2d)


def kernel(x, table, pos_enc):
  idx2 = x.reshape(NUM_WORKERS, CHUNKS_PER_WORKER, CHUNK)
  pos2d = pos_enc.reshape(MAX_LEN, D_MODEL)
  out = _run(table, idx2, pos2d)
  return out.reshape(BATCH, MAX_LEN, D_MODEL)


# speedup vs baseline: 3.0851x; 1.0215x over previous
import math

import jax
import jax.numpy as jnp
from jax import lax
from jax.experimental import pallas as pl
from jax.experimental.pallas import tpu as pltpu
from jax.experimental.pallas import tpu_sc as plsc

BATCH = 1024
MAX_LEN = 200
D_MODEL = 128
HALF = MAX_LEN // 2
NUM_WORKERS = 32
SEQ_PER_WORKER = BATCH // NUM_WORKERS
NBUF = 3
MAIN_CHUNKS = SEQ_PER_WORKER - (SEQ_PER_WORKER % NBUF)
SCALE = math.sqrt(float(D_MODEL))
LANES = 16
GROUPS = D_MODEL // LANES


def _sc_kernel(x_hbm, table_hbm, pos_hbm, out_hbm, idx_v, rows_v0, rows_v1,
               rows_v2, pos_v, gsem0, gsem1, gsem2, osem0, osem1, osem2,
               psem):
  wid = lax.axis_index("c") * 16 + lax.axis_index("s")
  base = wid * SEQ_PER_WORKER

  rows = (rows_v0, rows_v1, rows_v2)
  gsems = (gsem0, gsem1, gsem2)
  osems = (osem0, osem1, osem2)

  pltpu.sync_copy(x_hbm.at[wid], idx_v)

  def fire(c, buf):
    pltpu.async_copy(table_hbm.at[idx_v.at[2 * c]],
                     rows[buf].at[pl.ds(0, HALF)], gsems[buf])
    pltpu.async_copy(table_hbm.at[idx_v.at[2 * c + 1]],
                     rows[buf].at[pl.ds(HALF, HALF)], gsems[buf])

  def wait_gather(c, buf):
    pltpu.make_async_copy(table_hbm.at[idx_v.at[2 * c]],
                          rows[buf].at[pl.ds(0, HALF)], gsems[buf]).wait()
    pltpu.make_async_copy(table_hbm.at[idx_v.at[2 * c + 1]],
                          rows[buf].at[pl.ds(HALF, HALF)], gsems[buf]).wait()

  def wait_out(c, buf):
    pltpu.make_async_copy(rows[buf], out_hbm.at[base + c],
                          osems[buf]).wait()

  fire(0, 0)

  pltpu.async_copy(pos_hbm, pos_v, psem).wait()

  scale = jnp.float32(SCALE)

  def chunk_body(c, b):
    cur = rows[b]
    nb = (b + 1) % NBUF

    @pl.when(c + 1 < SEQ_PER_WORKER)
    def _():
      @pl.when(c >= 2)
      def _():
        wait_out(c - 2, nb)

      fire(c + 1, nb)

    wait_gather(c, b)

    @plsc.parallel_loop(0, MAX_LEN, unroll=2)
    def _(i):
      for k in range(GROUPS):
        sl = pl.ds(k * LANES, LANES)
        cur[i, sl] = cur[i, sl] * scale + pos_v[i, sl]

    pltpu.async_copy(cur, out_hbm.at[base + c], osems[b])

  def triple(ct, _):
    for b in range(NBUF):
      chunk_body(ct * NBUF + b, b)
    return 0

  lax.fori_loop(0, MAIN_CHUNKS // NBUF, triple, 0)

  for c in range(MAIN_CHUNKS, SEQ_PER_WORKER):
    chunk_body(jnp.int32(c), c % NBUF)

  for j in range(SEQ_PER_WORKER - NBUF, SEQ_PER_WORKER):
    wait_out(j, j % NBUF)


@jax.jit
def _run(x, table, pos2d):
  mesh = plsc.VectorSubcoreMesh(core_axis_name="c", subcore_axis_name="s")
  f = pl.kernel(
      _sc_kernel,
      out_type=jax.ShapeDtypeStruct((BATCH, MAX_LEN, D_MODEL), jnp.float32),
      mesh=mesh,
      scratch_types=[
          pltpu.VMEM((2 * SEQ_PER_WORKER, HALF), jnp.int32),
          pltpu.VMEM((MAX_LEN, D_MODEL), jnp.float32),
          pltpu.VMEM((MAX_LEN, D_MODEL), jnp.float32),
          pltpu.VMEM((MAX_LEN, D_MODEL), jnp.float32),
          pltpu.VMEM((MAX_LEN, D_MODEL), jnp.float32),
          pltpu.SemaphoreType.DMA,
          pltpu.SemaphoreType.DMA,
          pltpu.SemaphoreType.DMA,
          pltpu.SemaphoreType.DMA,
          pltpu.SemaphoreType.DMA,
          pltpu.SemaphoreType.DMA,
          pltpu.SemaphoreType.DMA,
      ],
  )
  return f(x, table, pos2d)


def kernel(x, table, pos_enc):
  idx3 = x.reshape(NUM_WORKERS, 2 * SEQ_PER_WORKER, HALF)
  pos2d = pos_enc.reshape(MAX_LEN, D_MODEL)
  return _run(idx3, table, pos2d)


# speedup vs baseline: 3.0918x; 1.0022x over previous
import math

import jax
import jax.numpy as jnp
from jax import lax
from jax.experimental import pallas as pl
from jax.experimental.pallas import tpu as pltpu
from jax.experimental.pallas import tpu_sc as plsc

BATCH = 1024
MAX_LEN = 200
D_MODEL = 128
HALF = MAX_LEN // 2
NUM_WORKERS = 32
SEQ_PER_WORKER = BATCH // NUM_WORKERS
NBUF = 3
MAIN_CHUNKS = SEQ_PER_WORKER - (SEQ_PER_WORKER % NBUF)
SCALE = math.sqrt(float(D_MODEL))
LANES = 16
GROUPS = D_MODEL // LANES


def _sc_kernel(x_hbm, table_hbm, pos_hbm, out_hbm, idx_v, rows_v0, rows_v1,
               rows_v2, pos_v, gsem0, gsem1, gsem2, osem0, osem1, osem2,
               psem):
  wid = lax.axis_index("c") * 16 + lax.axis_index("s")
  base = wid * SEQ_PER_WORKER

  rows = (rows_v0, rows_v1, rows_v2)
  gsems = (gsem0, gsem1, gsem2)
  osems = (osem0, osem1, osem2)

  pltpu.sync_copy(x_hbm.at[wid], idx_v)

  def fire(c, buf):
    pltpu.async_copy(table_hbm.at[idx_v.at[2 * c]],
                     rows[buf].at[pl.ds(0, HALF)], gsems[buf])
    pltpu.async_copy(table_hbm.at[idx_v.at[2 * c + 1]],
                     rows[buf].at[pl.ds(HALF, HALF)], gsems[buf])

  def wait_gather(c, buf):
    pltpu.make_async_copy(table_hbm.at[idx_v.at[2 * c]],
                          rows[buf].at[pl.ds(0, HALF)], gsems[buf]).wait()
    pltpu.make_async_copy(table_hbm.at[idx_v.at[2 * c + 1]],
                          rows[buf].at[pl.ds(HALF, HALF)], gsems[buf]).wait()

  def wait_out(c, buf):
    pltpu.make_async_copy(rows[buf], out_hbm.at[base + c],
                          osems[buf]).wait()

  fire(0, 0)

  pltpu.async_copy(pos_hbm, pos_v, psem).wait()

  scale = jnp.float32(SCALE)

  def chunk_body(c, b):
    cur = rows[b]
    nb = (b + 1) % NBUF

    @pl.when(c + 1 < SEQ_PER_WORKER)
    def _():
      @pl.when(c >= 2)
      def _():
        wait_out(c - 2, nb)

      fire(c + 1, nb)

    wait_gather(c, b)

    @plsc.parallel_loop(0, MAX_LEN, unroll=4)
    def _(i):
      for k in range(GROUPS):
        sl = pl.ds(k * LANES, LANES)
        cur[i, sl] = cur[i, sl] * scale + pos_v[i, sl]

    pltpu.async_copy(cur, out_hbm.at[base + c], osems[b])

  def triple(ct, _):
    for b in range(NBUF):
      chunk_body(ct * NBUF + b, b)
    return 0

  lax.fori_loop(0, MAIN_CHUNKS // NBUF, triple, 0)

  for c in range(MAIN_CHUNKS, SEQ_PER_WORKER):
    chunk_body(jnp.int32(c), c % NBUF)

  for j in range(SEQ_PER_WORKER - NBUF, SEQ_PER_WORKER):
    wait_out(j, j % NBUF)


@jax.jit
def _run(x, table, pos2d):
  mesh = plsc.VectorSubcoreMesh(core_axis_name="c", subcore_axis_name="s")
  f = pl.kernel(
      _sc_kernel,
      out_type=jax.ShapeDtypeStruct((BATCH, MAX_LEN, D_MODEL), jnp.float32),
      mesh=mesh,
      scratch_types=[
          pltpu.VMEM((2 * SEQ_PER_WORKER, HALF), jnp.int32),
          pltpu.VMEM((MAX_LEN, D_MODEL), jnp.float32),
          pltpu.VMEM((MAX_LEN, D_MODEL), jnp.float32),
          pltpu.VMEM((MAX_LEN, D_MODEL), jnp.float32),
          pltpu.VMEM((MAX_LEN, D_MODEL), jnp.float32),
          pltpu.SemaphoreType.DMA,
          pltpu.SemaphoreType.DMA,
          pltpu.SemaphoreType.DMA,
          pltpu.SemaphoreType.DMA,
          pltpu.SemaphoreType.DMA,
          pltpu.SemaphoreType.DMA,
          pltpu.SemaphoreType.DMA,
      ],
  )
  return f(x, table, pos2d)


def kernel(x, table, pos_enc):
  idx3 = x.reshape(NUM_WORKERS, 2 * SEQ_PER_WORKER, HALF)
  pos2d = pos_enc.reshape(MAX_LEN, D_MODEL)
  return _run(idx3, table, pos2d)


# speedup vs baseline: 3.1094x; 1.0057x over previous
import math

import jax
import jax.numpy as jnp
from jax import lax
from jax.experimental import pallas as pl
from jax.experimental.pallas import tpu as pltpu
from jax.experimental.pallas import tpu_sc as plsc

BATCH = 1024
MAX_LEN = 200
D_MODEL = 128
SPLIT_A = 104
SPLIT_B = MAX_LEN - SPLIT_A
NUM_WORKERS = 32
SEQ_PER_WORKER = BATCH // NUM_WORKERS
NBUF = 3
MAIN_CHUNKS = SEQ_PER_WORKER - (SEQ_PER_WORKER % NBUF)
SCALE = math.sqrt(float(D_MODEL))
LANES = 16
GROUPS = D_MODEL // LANES


def _sc_kernel(x_hbm, table_hbm, pos_hbm, out_hbm, idx_v, rows_v0, rows_v1,
               rows_v2, pos_v, gsem0, gsem1, gsem2, osem0, osem1, osem2,
               psem):
  wid = lax.axis_index("c") * 16 + lax.axis_index("s")
  base = wid * SEQ_PER_WORKER

  rows = (rows_v0, rows_v1, rows_v2)
  gsems = (gsem0, gsem1, gsem2)
  osems = (osem0, osem1, osem2)

  pltpu.sync_copy(x_hbm.at[wid], idx_v)

  def fire(c, buf):
    pltpu.async_copy(table_hbm.at[idx_v.at[pl.ds(c * MAX_LEN, SPLIT_A)]],
                     rows[buf].at[pl.ds(0, SPLIT_A)], gsems[buf])
    pltpu.async_copy(
        table_hbm.at[idx_v.at[pl.ds(c * MAX_LEN + SPLIT_A, SPLIT_B)]],
        rows[buf].at[pl.ds(SPLIT_A, SPLIT_B)], gsems[buf])

  def wait_gather(c, buf):
    pltpu.make_async_copy(table_hbm.at[idx_v.at[pl.ds(c * MAX_LEN, SPLIT_A)]],
                          rows[buf].at[pl.ds(0, SPLIT_A)], gsems[buf]).wait()
    pltpu.make_async_copy(
        table_hbm.at[idx_v.at[pl.ds(c * MAX_LEN + SPLIT_A, SPLIT_B)]],
        rows[buf].at[pl.ds(SPLIT_A, SPLIT_B)], gsems[buf]).wait()

  def wait_out(c, buf):
    pltpu.make_async_copy(rows[buf], out_hbm.at[base + c],
                          osems[buf]).wait()

  fire(0, 0)

  pltpu.async_copy(pos_hbm, pos_v, psem).wait()

  scale = jnp.float32(SCALE)

  def chunk_body(c, b):
    cur = rows[b]
    nb = (b + 1) % NBUF

    @pl.when(c + 1 < SEQ_PER_WORKER)
    def _():
      @pl.when(c >= 2)
      def _():
        wait_out(c - 2, nb)

      fire(c + 1, nb)

    wait_gather(c, b)

    @plsc.parallel_loop(0, MAX_LEN, unroll=2)
    def _(i):
      for k in range(GROUPS):
        sl = pl.ds(k * LANES, LANES)
        cur[i, sl] = cur[i, sl] * scale + pos_v[i, sl]

    pltpu.async_copy(cur, out_hbm.at[base + c], osems[b])

  def triple(ct, _):
    for b in range(NBUF):
      chunk_body(ct * NBUF + b, b)
    return 0

  lax.fori_loop(0, MAIN_CHUNKS // NBUF, triple, 0)

  for c in range(MAIN_CHUNKS, SEQ_PER_WORKER):
    chunk_body(jnp.int32(c), c % NBUF)

  for j in range(SEQ_PER_WORKER - NBUF, SEQ_PER_WORKER):
    wait_out(j, j % NBUF)


@jax.jit
def _run(x, table, pos2d):
  mesh = plsc.VectorSubcoreMesh(core_axis_name="c", subcore_axis_name="s")
  f = pl.kernel(
      _sc_kernel,
      out_type=jax.ShapeDtypeStruct((BATCH, MAX_LEN, D_MODEL), jnp.float32),
      mesh=mesh,
      scratch_types=[
          pltpu.VMEM((SEQ_PER_WORKER * MAX_LEN,), jnp.int32),
          pltpu.VMEM((MAX_LEN, D_MODEL), jnp.float32),
          pltpu.VMEM((MAX_LEN, D_MODEL), jnp.float32),
          pltpu.VMEM((MAX_LEN, D_MODEL), jnp.float32),
          pltpu.VMEM((MAX_LEN, D_MODEL), jnp.float32),
          pltpu.SemaphoreType.DMA,
          pltpu.SemaphoreType.DMA,
          pltpu.SemaphoreType.DMA,
          pltpu.SemaphoreType.DMA,
          pltpu.SemaphoreType.DMA,
          pltpu.SemaphoreType.DMA,
          pltpu.SemaphoreType.DMA,
      ],
  )
  return f(x, table, pos2d)


def kernel(x, table, pos_enc):
  idx2 = x.reshape(NUM_WORKERS, SEQ_PER_WORKER * MAX_LEN)
  pos2d = pos_enc.reshape(MAX_LEN, D_MODEL)
  return _run(idx2, table, pos2d)
